# initial kernel scaffold (unmeasured)
import jax
import jax.numpy as jnp
from jax import lax
from jax.experimental import pallas as pl
from jax.experimental.pallas import tpu as pltpu

N_DEV = 4
M = 4096
N = 2048
M_CHUNK = M // N_DEV


def kernel(x, w_mat, scale_x, scale_w):
    k_per = x.shape[1]
    assert x.shape == (M, k_per)
    assert w_mat.shape == (k_per, N)

    def body(x_ref, w_ref, sx_ref, sw_ref, out_ref, rbuf, send_sems, recv_sems):
        my = lax.axis_index("i")
        left = (my + N_DEV - 1) % N_DEV
        right = (my + 1) % N_DEV

        barrier_sem = pltpu.get_barrier_semaphore()
        for nbr in (left, right):
            pl.semaphore_signal(
                barrier_sem, inc=1,
                device_id=(nbr,), device_id_type=pl.DeviceIdType.MESH,
            )
        pl.semaphore_wait(barrier_sem, 2)

        scale = sx_ref[0] * sw_ref[0]
        for c in range(N_DEV):
            acc = jnp.dot(
                x_ref[pl.ds(c * M_CHUNK, M_CHUNK), :],
                w_ref[:, :],
                preferred_element_type=jnp.int32,
            )
            out_ref[pl.ds(c * M_CHUNK, M_CHUNK), :] = (
                acc.astype(jnp.float32) * scale
            )

        def hop(h, src_chunk):
            slot = h % 4
            rdma = pltpu.make_async_remote_copy(
                src_ref=out_ref.at[pl.ds(src_chunk * M_CHUNK, M_CHUNK), :],
                dst_ref=rbuf.at[slot],
                send_sem=send_sems.at[h],
                recv_sem=recv_sems.at[h],
                device_id=(right,),
                device_id_type=pl.DeviceIdType.MESH,
            )
            rdma.start()
            rdma.wait()
            return slot

        for s in range(N_DEV - 1):
            send_chunk = (my + N_DEV - s) % N_DEV
            slot = hop(s, send_chunk)
            acc_chunk = (my + N_DEV - 1 - s) % N_DEV
            row = pl.ds(acc_chunk * M_CHUNK, M_CHUNK)
            out_ref[row, :] = out_ref[row, :] + rbuf[slot]

        for t in range(N_DEV - 1):
            send_chunk = (my + N_DEV + 1 - t) % N_DEV
            slot = hop(N_DEV - 1 + t, send_chunk)
            store_chunk = (my + N_DEV - t) % N_DEV
            out_ref[pl.ds(store_chunk * M_CHUNK, M_CHUNK), :] = rbuf[slot]

    n_hops = 2 * (N_DEV - 1)
    return pl.pallas_call(
        body,
        out_shape=jax.ShapeDtypeStruct((M, N), jnp.float32),
        in_specs=[
            pl.BlockSpec(memory_space=pltpu.VMEM),
            pl.BlockSpec(memory_space=pltpu.VMEM),
            pl.BlockSpec(memory_space=pltpu.SMEM),
            pl.BlockSpec(memory_space=pltpu.SMEM),
        ],
        out_specs=pl.BlockSpec(memory_space=pltpu.VMEM),
        scratch_shapes=[
            pltpu.VMEM((4, M_CHUNK, N), jnp.float32),
            pltpu.SemaphoreType.DMA((n_hops,)),
            pltpu.SemaphoreType.DMA((n_hops,)),
        ],
        compiler_params=pltpu.CompilerParams(collective_id=0),
    )(x, w_mat, scale_x, scale_w)


# baseline (device time: 623446 ns/iter reference)
import jax
import jax.numpy as jnp
from jax import lax
from jax.experimental import pallas as pl
from jax.experimental.pallas import tpu as pltpu

N_DEV = 4
M = 4096
N = 2048
M_CHUNK = M // N_DEV
N_HALF = N // 2


def kernel(x, w_mat, scale_x, scale_w):
    k_per = x.shape[1]
    assert x.shape == (M, k_per)
    assert w_mat.shape == (k_per, N)

    def body(x_ref, w_ref, sx_ref, sw_ref, out_ref, rbuf, send_sems, recv_sems):
        my = lax.axis_index("i")
        left = (my + N_DEV - 1) % N_DEV
        right = (my + 1) % N_DEV

        barrier_sem = pltpu.get_barrier_semaphore()
        for nbr in (left, right):
            pl.semaphore_signal(
                barrier_sem, inc=1,
                device_id=(nbr,), device_id_type=pl.DeviceIdType.MESH,
            )
        pl.semaphore_wait(barrier_sem, 2)

        scale = sx_ref[0] * sw_ref[0]

        def gemm_tile(i, _):
            c = i // 2
            half = i % 2
            rows = pl.ds(c * M_CHUNK, M_CHUNK)
            cols = pl.ds(half * N_HALF, N_HALF)
            acc = jnp.dot(
                x_ref[rows, :],
                w_ref[:, cols],
                preferred_element_type=jnp.int32,
            )
            out_ref[rows, cols] = acc.astype(jnp.float32) * scale
            return 0

        lax.fori_loop(0, 2 * N_DEV, gemm_tile, 0)

        def start_hop(h, src_chunk, cols):
            rdma = pltpu.make_async_remote_copy(
                src_ref=out_ref.at[pl.ds(src_chunk * M_CHUNK, M_CHUNK), cols],
                dst_ref=rbuf.at[h % 4],
                send_sem=send_sems.at[h],
                recv_sem=recv_sems.at[h],
                device_id=(right,),
                device_id_type=pl.DeviceIdType.MESH,
            )
            rdma.start()
            rdma.wait()

        def ring_allreduce(half, hop_base):
            cols = pl.ds(half * N_HALF, N_HALF)

            def rs_hop(s, _):
                h = hop_base + s
                start_hop(h, (my + N_DEV - s) % N_DEV, cols)
                acc_chunk = (my + N_DEV - 1 - s) % N_DEV
                rows = pl.ds(acc_chunk * M_CHUNK, M_CHUNK)
                out_ref[rows, cols] = out_ref[rows, cols] + rbuf[h % 4]
                return 0

            lax.fori_loop(0, N_DEV - 1, rs_hop, 0)

            def ag_hop(t, _):
                h = hop_base + N_DEV - 1 + t
                start_hop(h, (my + N_DEV + 1 - t) % N_DEV, cols)
                store_chunk = (my + N_DEV - t) % N_DEV
                out_ref[pl.ds(store_chunk * M_CHUNK, M_CHUNK), cols] = rbuf[h % 4]
                return 0

            lax.fori_loop(0, N_DEV - 1, ag_hop, 0)

        ring_allreduce(0, 0)
        ring_allreduce(1, 2 * (N_DEV - 1))

    n_hops = 4 * (N_DEV - 1)
    return pl.pallas_call(
        body,
        out_shape=jax.ShapeDtypeStruct((M, N), jnp.float32),
        in_specs=[
            pl.BlockSpec(memory_space=pltpu.VMEM),
            pl.BlockSpec(memory_space=pltpu.VMEM),
            pl.BlockSpec(memory_space=pltpu.SMEM),
            pl.BlockSpec(memory_space=pltpu.SMEM),
        ],
        out_specs=pl.BlockSpec(memory_space=pltpu.VMEM),
        scratch_shapes=[
            pltpu.VMEM((4, M_CHUNK, N_HALF), jnp.float32),
            pltpu.SemaphoreType.DMA((n_hops,)),
            pltpu.SemaphoreType.DMA((n_hops,)),
        ],
        compiler_params=pltpu.CompilerParams(
            collective_id=0,
            vmem_limit_bytes=64 * 1024 * 1024,
        ),
    )(x, w_mat, scale_x, scale_w)


# device time: 329104 ns/iter; 1.8944x vs baseline; 1.8944x over previous
import jax
import jax.numpy as jnp
from jax import lax
from jax.experimental import pallas as pl
from jax.experimental.pallas import tpu as pltpu

N_DEV = 4
M = 4096
N = 2048
M_CHUNK = M // N_DEV
N_HALF = N // 2
N_HOPS = 2 * (N_DEV - 1)


def kernel(x, w_mat, scale_x, scale_w):
    k_per = x.shape[1]
    assert x.shape == (M, k_per)
    assert w_mat.shape == (k_per, N)

    def body(x_ref, w_ref, sx_ref, sw_ref, out_ref,
             rbuf_r, rbuf_l, send_r, recv_r, send_l, recv_l,
             cred_r, cred_l):
        my = lax.axis_index("i")
        left = (my + N_DEV - 1) % N_DEV
        right = (my + 1) % N_DEV
        cols0 = pl.ds(0, N_HALF)
        cols1 = pl.ds(N_HALF, N_HALF)

        barrier_sem = pltpu.get_barrier_semaphore()
        for nbr in (left, right):
            pl.semaphore_signal(
                barrier_sem, inc=1,
                device_id=(nbr,), device_id_type=pl.DeviceIdType.MESH,
            )
        pl.semaphore_wait(barrier_sem, 2)

        scale = sx_ref[0] * sw_ref[0]

        def gemm_tile(i, _):
            c = (my + i // 2) % N_DEV
            rows = pl.ds(c * M_CHUNK, M_CHUNK)
            cols = pl.ds((i % 2) * N_HALF, N_HALF)
            acc = jnp.dot(
                x_ref[rows, :],
                w_ref[:, cols],
                preferred_element_type=jnp.int32,
            )
            out_ref[rows, cols] = acc.astype(jnp.float32) * scale
            return 0

        def make_hop(h, slot):
            src_r = (my + 2 * N_HOPS - h) % N_DEV
            src_l = (my + h) % N_DEV
            rdma_r = pltpu.make_async_remote_copy(
                src_ref=out_ref.at[pl.ds(src_r * M_CHUNK, M_CHUNK), cols0],
                dst_ref=rbuf_r.at[slot],
                send_sem=send_r.at[h],
                recv_sem=recv_r.at[h],
                device_id=(right,),
                device_id_type=pl.DeviceIdType.MESH,
            )
            rdma_l = pltpu.make_async_remote_copy(
                src_ref=out_ref.at[pl.ds(src_l * M_CHUNK, M_CHUNK), cols1],
                dst_ref=rbuf_l.at[slot],
                send_sem=send_l.at[h],
                recv_sem=recv_l.at[h],
                device_id=(left,),
                device_id_type=pl.DeviceIdType.MESH,
            )
            return rdma_r, rdma_l

        def merge(h, slot):
            rows_r = pl.ds(((my + 2 * N_HOPS - h - 1) % N_DEV) * M_CHUNK, M_CHUNK)
            rows_l = pl.ds(((my + h + 1) % N_DEV) * M_CHUNK, M_CHUNK)

            @pl.when(h < N_DEV - 1)
            def _():
                out_ref[rows_r, cols0] = out_ref[rows_r, cols0] + rbuf_r[slot]
                out_ref[rows_l, cols1] = out_ref[rows_l, cols1] + rbuf_l[slot]

            @pl.when(h >= N_DEV - 1)
            def _():
                out_ref[rows_r, cols0] = rbuf_r[slot]
                out_ref[rows_l, cols1] = rbuf_l[slot]

            @pl.when(h < N_HOPS - 2)
            def _():
                pl.semaphore_signal(
                    cred_r, inc=1,
                    device_id=(left,), device_id_type=pl.DeviceIdType.MESH,
                )
                pl.semaphore_signal(
                    cred_l, inc=1,
                    device_id=(right,), device_id_type=pl.DeviceIdType.MESH,
                )

        lax.fori_loop(0, 2, gemm_tile, 0)
        rdma0_r, rdma0_l = make_hop(0, 0)
        rdma0_r.start()
        rdma0_l.start()
        lax.fori_loop(2, 2 * N_DEV, gemm_tile, 0)
        rdma0_r.wait()
        rdma0_l.wait()
        merge(0, 0)

        def hop_body(h, _):
            slot = h % 2

            @pl.when(h >= 2)
            def _():
                pl.semaphore_wait(cred_r, 1)
                pl.semaphore_wait(cred_l, 1)

            rdma_r, rdma_l = make_hop(h, slot)
            rdma_r.start()
            rdma_l.start()
            rdma_r.wait()
            rdma_l.wait()
            merge(h, slot)
            return 0

        lax.fori_loop(1, N_HOPS, hop_body, 0)

    return pl.pallas_call(
        body,
        out_shape=jax.ShapeDtypeStruct((M, N), jnp.float32),
        in_specs=[
            pl.BlockSpec(memory_space=pltpu.VMEM),
            pl.BlockSpec(memory_space=pltpu.VMEM),
            pl.BlockSpec(memory_space=pltpu.SMEM),
            pl.BlockSpec(memory_space=pltpu.SMEM),
        ],
        out_specs=pl.BlockSpec(memory_space=pltpu.VMEM),
        scratch_shapes=[
            pltpu.VMEM((2, M_CHUNK, N_HALF), jnp.float32),
            pltpu.VMEM((2, M_CHUNK, N_HALF), jnp.float32),
            pltpu.SemaphoreType.DMA((N_HOPS,)),
            pltpu.SemaphoreType.DMA((N_HOPS,)),
            pltpu.SemaphoreType.DMA((N_HOPS,)),
            pltpu.SemaphoreType.DMA((N_HOPS,)),
            pltpu.SemaphoreType.REGULAR,
            pltpu.SemaphoreType.REGULAR,
        ],
        compiler_params=pltpu.CompilerParams(
            collective_id=0,
            vmem_limit_bytes=64 * 1024 * 1024,
        ),
    )(x, w_mat, scale_x, scale_w)


# device time: 327248 ns/iter; 1.9051x vs baseline; 1.0057x over previous
import jax
import jax.numpy as jnp
from jax import lax
from jax.experimental import pallas as pl
from jax.experimental.pallas import tpu as pltpu

N_DEV = 4
M = 4096
N = 2048
M_CHUNK = M // N_DEV
N_HALF = N // 2
N_HOPS = 2 * (N_DEV - 1)


def kernel(x, w_mat, scale_x, scale_w):
    k_per = x.shape[1]
    assert x.shape == (M, k_per)
    assert w_mat.shape == (k_per, N)

    def body(x_ref, w_ref, sx_ref, sw_ref, out_ref,
             rbuf_r, rbuf_l, send_r, recv_r, send_l, recv_l,
             cred_r, cred_l):
        my = lax.axis_index("i")
        left = (my + N_DEV - 1) % N_DEV
        right = (my + 1) % N_DEV
        cols0 = pl.ds(0, N_HALF)
        cols1 = pl.ds(N_HALF, N_HALF)

        barrier_sem = pltpu.get_barrier_semaphore()
        for nbr in (left, right):
            pl.semaphore_signal(
                barrier_sem, inc=1,
                device_id=(nbr,), device_id_type=pl.DeviceIdType.MESH,
            )
        pl.semaphore_wait(barrier_sem, 2)

        scale = sx_ref[0] * sw_ref[0]

        def gemm_tile(i, _):
            c = (my + i // 2) % N_DEV
            rows = pl.ds(c * M_CHUNK, M_CHUNK)
            cols = pl.ds((i % 2) * N_HALF, N_HALF)
            acc = jnp.dot(
                x_ref[rows, :],
                w_ref[:, cols],
                preferred_element_type=jnp.int32,
            )
            out_ref[rows, cols] = acc.astype(jnp.float32) * scale
            return 0

        def make_hop(h, slot, is_ag):
            src_r = (my + 2 * N_HOPS - h) % N_DEV
            src_l = (my + h) % N_DEV
            rows_r = pl.ds(src_r * M_CHUNK, M_CHUNK)
            rows_l = pl.ds(src_l * M_CHUNK, M_CHUNK)
            rdma_r = pltpu.make_async_remote_copy(
                src_ref=out_ref.at[rows_r, cols0],
                dst_ref=out_ref.at[rows_r, cols0] if is_ag else rbuf_r.at[slot],
                send_sem=send_r.at[h],
                recv_sem=recv_r.at[h],
                device_id=(right,),
                device_id_type=pl.DeviceIdType.MESH,
            )
            rdma_l = pltpu.make_async_remote_copy(
                src_ref=out_ref.at[rows_l, cols1],
                dst_ref=out_ref.at[rows_l, cols1] if is_ag else rbuf_l.at[slot],
                send_sem=send_l.at[h],
                recv_sem=recv_l.at[h],
                device_id=(left,),
                device_id_type=pl.DeviceIdType.MESH,
            )
            return rdma_r, rdma_l

        def merge(h, slot):

            @pl.when(h < N_DEV - 1)
            def _():
                rows_r = pl.ds(
                    ((my + 2 * N_HOPS - h - 1) % N_DEV) * M_CHUNK, M_CHUNK
                )
                rows_l = pl.ds(((my + h + 1) % N_DEV) * M_CHUNK, M_CHUNK)
                out_ref[rows_r, cols0] = out_ref[rows_r, cols0] + rbuf_r[slot]
                out_ref[rows_l, cols1] = out_ref[rows_l, cols1] + rbuf_l[slot]

            @pl.when(h < N_HOPS - 2)
            def _():
                pl.semaphore_signal(
                    cred_r, inc=1,
                    device_id=(left,), device_id_type=pl.DeviceIdType.MESH,
                )
                pl.semaphore_signal(
                    cred_l, inc=1,
                    device_id=(right,), device_id_type=pl.DeviceIdType.MESH,
                )

        lax.fori_loop(0, 2, gemm_tile, 0)
        rdma0_r, rdma0_l = make_hop(0, 0, False)
        rdma0_r.start()
        rdma0_l.start()
        lax.fori_loop(2, 2 * N_DEV, gemm_tile, 0)
        rdma0_r.wait()
        rdma0_l.wait()
        merge(0, 0)

        def hop_body(h, _, is_ag):
            slot = h % 2

            @pl.when(h >= 2)
            def _():
                pl.semaphore_wait(cred_r, 1)
                pl.semaphore_wait(cred_l, 1)

            rdma_r, rdma_l = make_hop(h, slot, is_ag)
            rdma_r.start()
            rdma_l.start()
            rdma_r.wait()
            rdma_l.wait()
            merge(h, slot)
            return 0

        lax.fori_loop(1, N_DEV - 1, lambda h, c: hop_body(h, c, False), 0)
        lax.fori_loop(N_DEV - 1, N_HOPS, lambda h, c: hop_body(h, c, True), 0)

    return pl.pallas_call(
        body,
        out_shape=jax.ShapeDtypeStruct((M, N), jnp.float32),
        in_specs=[
            pl.BlockSpec(memory_space=pltpu.VMEM),
            pl.BlockSpec(memory_space=pltpu.VMEM),
            pl.BlockSpec(memory_space=pltpu.SMEM),
            pl.BlockSpec(memory_space=pltpu.SMEM),
        ],
        out_specs=pl.BlockSpec(memory_space=pltpu.VMEM),
        scratch_shapes=[
            pltpu.VMEM((2, M_CHUNK, N_HALF), jnp.float32),
            pltpu.VMEM((2, M_CHUNK, N_HALF), jnp.float32),
            pltpu.SemaphoreType.DMA((N_HOPS,)),
            pltpu.SemaphoreType.DMA((N_HOPS,)),
            pltpu.SemaphoreType.DMA((N_HOPS,)),
            pltpu.SemaphoreType.DMA((N_HOPS,)),
            pltpu.SemaphoreType.REGULAR,
            pltpu.SemaphoreType.REGULAR,
        ],
        compiler_params=pltpu.CompilerParams(
            collective_id=0,
            vmem_limit_bytes=64 * 1024 * 1024,
        ),
    )(x, w_mat, scale_x, scale_w)


# device time: 317461 ns/iter; 1.9639x vs baseline; 1.0308x over previous
import jax
import jax.numpy as jnp
from jax import lax
from jax.experimental import pallas as pl
from jax.experimental.pallas import tpu as pltpu

N_DEV = 4
M = 4096
N = 2048
M_CHUNK = M // N_DEV
M_SUB = M_CHUNK // 2
N_HALF = N // 2
N_HOPS = 2 * (N_DEV - 1)


def kernel(x, w_mat, scale_x, scale_w):
    k_per = x.shape[1]
    assert x.shape == (M, k_per)
    assert w_mat.shape == (k_per, N)

    def body(x_ref, w_ref, sx_ref, sw_ref, out_ref,
             rbuf_ra, rbuf_rb, rbuf_la, rbuf_lb,
             send_ra, recv_ra, send_rb, recv_rb,
             send_la, recv_la, send_lb, recv_lb,
             cred_ra, cred_rb, cred_la, cred_lb):
        my = lax.axis_index("i")
        left = (my + N_DEV - 1) % N_DEV
        right = (my + 1) % N_DEV
        cols0 = pl.ds(0, N_HALF)
        cols1 = pl.ds(N_HALF, N_HALF)

        barrier_sem = pltpu.get_barrier_semaphore()
        for nbr in (left, right):
            pl.semaphore_signal(
                barrier_sem, inc=1,
                device_id=(nbr,), device_id_type=pl.DeviceIdType.MESH,
            )
        pl.semaphore_wait(barrier_sem, 2)

        scale = sx_ref[0] * sw_ref[0]

        def gemm_tile(i, _):
            c = (my + i // 2) % N_DEV
            rows = pl.ds(c * M_CHUNK, M_CHUNK)
            cols = pl.ds((i % 2) * N_HALF, N_HALF)
            acc = jnp.dot(
                x_ref[rows, :],
                w_ref[:, cols],
                preferred_element_type=jnp.int32,
            )
            out_ref[rows, cols] = acc.astype(jnp.float32) * scale
            return 0

        def sub_rows(chunk, sub):
            return pl.ds(chunk * M_CHUNK + sub * M_SUB, M_SUB)

        def make_hop(h, sub, is_ag):
            slot = h % 2
            src_r = (my + 2 * N_HOPS - h) % N_DEV
            src_l = (my + h) % N_DEV
            rows_r = sub_rows(src_r, sub)
            rows_l = sub_rows(src_l, sub)
            rbuf_r = rbuf_rb if sub else rbuf_ra
            rbuf_l = rbuf_lb if sub else rbuf_la
            rdma_r = pltpu.make_async_remote_copy(
                src_ref=out_ref.at[rows_r, cols0],
                dst_ref=out_ref.at[rows_r, cols0] if is_ag else rbuf_r.at[slot],
                send_sem=(send_rb if sub else send_ra).at[h],
                recv_sem=(recv_rb if sub else recv_ra).at[h],
                device_id=(right,),
                device_id_type=pl.DeviceIdType.MESH,
            )
            rdma_l = pltpu.make_async_remote_copy(
                src_ref=out_ref.at[rows_l, cols1],
                dst_ref=out_ref.at[rows_l, cols1] if is_ag else rbuf_l.at[slot],
                send_sem=(send_lb if sub else send_la).at[h],
                recv_sem=(recv_lb if sub else recv_la).at[h],
                device_id=(left,),
                device_id_type=pl.DeviceIdType.MESH,
            )
            return rdma_r, rdma_l

        def start_hop(h, sub, is_ag):
            rdma_r, rdma_l = make_hop(h, sub, is_ag)
            rdma_r.start()
            rdma_l.start()

        def finish_hop(h, sub, is_ag):
            rdma_r, rdma_l = make_hop(h, sub, is_ag)
            rdma_r.wait()
            rdma_l.wait()
            slot = h % 2
            rbuf_r = rbuf_rb if sub else rbuf_ra
            rbuf_l = rbuf_lb if sub else rbuf_la

            @pl.when(h < N_DEV - 1)
            def _():
                rows_r = sub_rows((my + 2 * N_HOPS - h - 1) % N_DEV, sub)
                rows_l = sub_rows((my + h + 1) % N_DEV, sub)
                out_ref[rows_r, cols0] = out_ref[rows_r, cols0] + rbuf_r[slot]
                out_ref[rows_l, cols1] = out_ref[rows_l, cols1] + rbuf_l[slot]

            @pl.when(h < N_HOPS - 2)
            def _():
                pl.semaphore_signal(
                    cred_rb if sub else cred_ra, inc=1,
                    device_id=(left,), device_id_type=pl.DeviceIdType.MESH,
                )
                pl.semaphore_signal(
                    cred_lb if sub else cred_la, inc=1,
                    device_id=(right,), device_id_type=pl.DeviceIdType.MESH,
                )

        def wait_credits(h, sub):
            @pl.when(h >= 2)
            def _():
                pl.semaphore_wait(cred_rb if sub else cred_ra, 1)
                pl.semaphore_wait(cred_lb if sub else cred_la, 1)

        lax.fori_loop(0, 2, gemm_tile, 0)
        start_hop(0, 0, False)
        start_hop(0, 1, False)
        lax.fori_loop(2, 2 * N_DEV, gemm_tile, 0)
        finish_hop(0, 0, False)

        def loop_body(h, _, a_ag, b_prev_ag):
            wait_credits(h, 0)
            start_hop(h, 0, a_ag)
            finish_hop(h - 1, 1, b_prev_ag)
            wait_credits(h, 1)
            start_hop(h, 1, a_ag)
            finish_hop(h, 0, a_ag)
            return 0

        ag = N_DEV - 1
        lax.fori_loop(1, ag, lambda h, c: loop_body(h, c, False, False), 0)
        loop_body(ag, 0, True, False)
        lax.fori_loop(ag + 1, N_HOPS, lambda h, c: loop_body(h, c, True, True), 0)

        finish_hop(N_HOPS - 1, 1, True)

    dma6 = pltpu.SemaphoreType.DMA((N_HOPS,))
    return pl.pallas_call(
        body,
        out_shape=jax.ShapeDtypeStruct((M, N), jnp.float32),
        in_specs=[
            pl.BlockSpec(memory_space=pltpu.VMEM),
            pl.BlockSpec(memory_space=pltpu.VMEM),
            pl.BlockSpec(memory_space=pltpu.SMEM),
            pl.BlockSpec(memory_space=pltpu.SMEM),
        ],
        out_specs=pl.BlockSpec(memory_space=pltpu.VMEM),
        scratch_shapes=[
            pltpu.VMEM((2, M_SUB, N_HALF), jnp.float32),
            pltpu.VMEM((2, M_SUB, N_HALF), jnp.float32),
            pltpu.VMEM((2, M_SUB, N_HALF), jnp.float32),
            pltpu.VMEM((2, M_SUB, N_HALF), jnp.float32),
            dma6, dma6,
            dma6, dma6,
            dma6, dma6,
            dma6, dma6,
            pltpu.SemaphoreType.REGULAR,
            pltpu.SemaphoreType.REGULAR,
            pltpu.SemaphoreType.REGULAR,
            pltpu.SemaphoreType.REGULAR,
        ],
        compiler_params=pltpu.CompilerParams(
            collective_id=0,
            vmem_limit_bytes=64 * 1024 * 1024,
        ),
    )(x, w_mat, scale_x, scale_w)


# device time: 316840 ns/iter; 1.9677x vs baseline; 1.0020x over previous
import jax
import jax.numpy as jnp
from jax import lax
from jax.experimental import pallas as pl
from jax.experimental.pallas import tpu as pltpu

N_DEV = 4
M = 4096
N = 2048
M_CHUNK = M // N_DEV
N_SUB = 4
M_SUB = M_CHUNK // N_SUB
N_HALF = N // 2
N_HOPS = 2 * (N_DEV - 1)


def kernel(x, w_mat, scale_x, scale_w):
    k_per = x.shape[1]
    assert x.shape == (M, k_per)
    assert w_mat.shape == (k_per, N)

    def body(x_ref, w_ref, sx_ref, sw_ref, out_ref, *scratch):
        rbuf_r = scratch[:N_SUB]
        rbuf_l = scratch[N_SUB:2 * N_SUB]
        dma = scratch[2 * N_SUB:2 * N_SUB + 4 * N_SUB]
        send_r = dma[0::4]
        recv_r = dma[1::4]
        send_l = dma[2::4]
        recv_l = dma[3::4]
        creds = scratch[2 * N_SUB + 4 * N_SUB:]
        cred_r = creds[0::2]
        cred_l = creds[1::2]

        my = lax.axis_index("i")
        left = (my + N_DEV - 1) % N_DEV
        right = (my + 1) % N_DEV
        cols0 = pl.ds(0, N_HALF)
        cols1 = pl.ds(N_HALF, N_HALF)

        barrier_sem = pltpu.get_barrier_semaphore()
        for nbr in (left, right):
            pl.semaphore_signal(
                barrier_sem, inc=1,
                device_id=(nbr,), device_id_type=pl.DeviceIdType.MESH,
            )
        pl.semaphore_wait(barrier_sem, 2)

        scale = sx_ref[0] * sw_ref[0]

        def gemm_tile(i, _):
            c = (my + i // 2) % N_DEV
            rows = pl.ds(c * M_CHUNK, M_CHUNK)
            cols = pl.ds((i % 2) * N_HALF, N_HALF)
            acc = jnp.dot(
                x_ref[rows, :],
                w_ref[:, cols],
                preferred_element_type=jnp.int32,
            )
            out_ref[rows, cols] = acc.astype(jnp.float32) * scale
            return 0

        def sub_rows(chunk, s):
            return pl.ds(chunk * M_CHUNK + s * M_SUB, M_SUB)

        def make_hop(h, s, is_ag):
            slot = h % 2
            rows_r = sub_rows((my + 2 * N_HOPS - h) % N_DEV, s)
            rows_l = sub_rows((my + h) % N_DEV, s)
            rdma_r = pltpu.make_async_remote_copy(
                src_ref=out_ref.at[rows_r, cols0],
                dst_ref=out_ref.at[rows_r, cols0] if is_ag else rbuf_r[s].at[slot],
                send_sem=send_r[s].at[slot],
                recv_sem=recv_r[s].at[slot],
                device_id=(right,),
                device_id_type=pl.DeviceIdType.MESH,
            )
            rdma_l = pltpu.make_async_remote_copy(
                src_ref=out_ref.at[rows_l, cols1],
                dst_ref=out_ref.at[rows_l, cols1] if is_ag else rbuf_l[s].at[slot],
                send_sem=send_l[s].at[slot],
                recv_sem=recv_l[s].at[slot],
                device_id=(left,),
                device_id_type=pl.DeviceIdType.MESH,
            )
            return rdma_r, rdma_l

        def start_hop(h, s, is_ag):
            rdma_r, rdma_l = make_hop(h, s, is_ag)
            rdma_r.start()
            rdma_l.start()

        def finish_hop(h, s, is_ag):
            rdma_r, rdma_l = make_hop(h, s, is_ag)
            rdma_r.wait()
            rdma_l.wait()
            slot = h % 2

            @pl.when(h < N_DEV - 1)
            def _():
                rows_r = sub_rows((my + 2 * N_HOPS - h - 1) % N_DEV, s)
                rows_l = sub_rows((my + h + 1) % N_DEV, s)
                out_ref[rows_r, cols0] = out_ref[rows_r, cols0] + rbuf_r[s][slot]
                out_ref[rows_l, cols1] = out_ref[rows_l, cols1] + rbuf_l[s][slot]

            @pl.when(h < N_HOPS - 2)
            def _():
                pl.semaphore_signal(
                    cred_r[s], inc=1,
                    device_id=(left,), device_id_type=pl.DeviceIdType.MESH,
                )
                pl.semaphore_signal(
                    cred_l[s], inc=1,
                    device_id=(right,), device_id_type=pl.DeviceIdType.MESH,
                )

        def wait_credits(h, s):
            @pl.when(h >= 2)
            def _():
                pl.semaphore_wait(cred_r[s], 1)
                pl.semaphore_wait(cred_l[s], 1)

        lax.fori_loop(0, 2, gemm_tile, 0)
        for s in range(N_SUB):
            start_hop(0, s, False)
        lax.fori_loop(2, 2 * N_DEV, gemm_tile, 0)
        for s in range(N_SUB - 1):
            finish_hop(0, s, False)

        def loop_body(h, _, is_ag, prev_ag):
            wait_credits(h, 0)
            start_hop(h, 0, is_ag)
            finish_hop(h - 1, N_SUB - 1, prev_ag)
            for s in range(1, N_SUB):
                wait_credits(h, s)
                start_hop(h, s, is_ag)
                finish_hop(h, s - 1, is_ag)
            return 0

        ag = N_DEV - 1
        lax.fori_loop(1, ag, lambda h, c: loop_body(h, c, False, False), 0)
        loop_body(ag, 0, True, False)
        lax.fori_loop(ag + 1, N_HOPS, lambda h, c: loop_body(h, c, True, True), 0)

        finish_hop(N_HOPS - 1, N_SUB - 1, True)

    dma_sems = pltpu.SemaphoreType.DMA((2,))
    return pl.pallas_call(
        body,
        out_shape=jax.ShapeDtypeStruct((M, N), jnp.float32),
        in_specs=[
            pl.BlockSpec(memory_space=pltpu.VMEM),
            pl.BlockSpec(memory_space=pltpu.VMEM),
            pl.BlockSpec(memory_space=pltpu.SMEM),
            pl.BlockSpec(memory_space=pltpu.SMEM),
        ],
        out_specs=pl.BlockSpec(memory_space=pltpu.VMEM),
        scratch_shapes=(
            [pltpu.VMEM((2, M_SUB, N_HALF), jnp.float32)] * (2 * N_SUB)
            + [dma_sems] * (4 * N_SUB)
            + [pltpu.SemaphoreType.REGULAR] * (2 * N_SUB)
        ),
        compiler_params=pltpu.CompilerParams(
            collective_id=0,
            vmem_limit_bytes=64 * 1024 * 1024,
        ),
    )(x, w_mat, scale_x, scale_w)


# device time: 316833 ns/iter; 1.9677x vs baseline; 1.0000x over previous
import jax
import jax.numpy as jnp
from jax import lax
from jax.experimental import pallas as pl
from jax.experimental.pallas import tpu as pltpu

N_DEV = 4
M = 4096
N = 2048
M_HALF = M // 2
M_CHUNK = M_HALF // N_DEV
N_SUB = 4
M_SUB = M_CHUNK // N_SUB
N_HOPS = 2 * (N_DEV - 1)


def kernel(x, w_mat, scale_x, scale_w):
    k_per = x.shape[1]
    assert x.shape == (M, k_per)
    assert w_mat.shape == (k_per, N)

    def body(x_ref, w_ref, sx_ref, sw_ref, out_ref, *scratch):
        rbuf_r = scratch[:N_SUB]
        rbuf_l = scratch[N_SUB:2 * N_SUB]
        dma = scratch[2 * N_SUB:2 * N_SUB + 4 * N_SUB]
        send_r = dma[0::4]
        recv_r = dma[1::4]
        send_l = dma[2::4]
        recv_l = dma[3::4]
        creds = scratch[2 * N_SUB + 4 * N_SUB:]
        cred_r = creds[0::2]
        cred_l = creds[1::2]

        my = lax.axis_index("i")
        left = (my + N_DEV - 1) % N_DEV
        right = (my + 1) % N_DEV

        barrier_sem = pltpu.get_barrier_semaphore()
        for nbr in (left, right):
            pl.semaphore_signal(
                barrier_sem, inc=1,
                device_id=(nbr,), device_id_type=pl.DeviceIdType.MESH,
            )
        pl.semaphore_wait(barrier_sem, 2)

        scale = sx_ref[0] * sw_ref[0]

        def gemm_block(i, _):
            blk = (i % 2) * N_DEV + (my + i // 2) % N_DEV
            rows = pl.ds(blk * M_CHUNK, M_CHUNK)
            acc = jnp.dot(
                x_ref[rows, :],
                w_ref[:, :],
                preferred_element_type=jnp.int32,
            )
            out_ref[rows, :] = acc.astype(jnp.float32) * scale
            return 0

        def rows_r(chunk, s):
            return pl.ds(chunk * M_CHUNK + s * M_SUB, M_SUB)

        def rows_l(chunk, s):
            return pl.ds(M_HALF + chunk * M_CHUNK + s * M_SUB, M_SUB)

        def make_hop(h, s, is_ag):
            slot = h % 2
            rr = rows_r((my + 2 * N_HOPS - h) % N_DEV, s)
            rl = rows_l((my + h) % N_DEV, s)
            rdma_r = pltpu.make_async_remote_copy(
                src_ref=out_ref.at[rr, :],
                dst_ref=out_ref.at[rr, :] if is_ag else rbuf_r[s].at[slot],
                send_sem=send_r[s].at[slot],
                recv_sem=recv_r[s].at[slot],
                device_id=(right,),
                device_id_type=pl.DeviceIdType.MESH,
            )
            rdma_l = pltpu.make_async_remote_copy(
                src_ref=out_ref.at[rl, :],
                dst_ref=out_ref.at[rl, :] if is_ag else rbuf_l[s].at[slot],
                send_sem=send_l[s].at[slot],
                recv_sem=recv_l[s].at[slot],
                device_id=(left,),
                device_id_type=pl.DeviceIdType.MESH,
            )
            return rdma_r, rdma_l

        def start_hop(h, s, is_ag):
            rdma_r, rdma_l = make_hop(h, s, is_ag)
            rdma_r.start()
            rdma_l.start()

        def finish_hop(h, s, is_ag):
            rdma_r, rdma_l = make_hop(h, s, is_ag)
            rdma_r.wait()
            rdma_l.wait()
            slot = h % 2

            @pl.when(h < N_DEV - 1)
            def _():
                rr = rows_r((my + 2 * N_HOPS - h - 1) % N_DEV, s)
                rl = rows_l((my + h + 1) % N_DEV, s)
                out_ref[rr, :] = out_ref[rr, :] + rbuf_r[s][slot]
                out_ref[rl, :] = out_ref[rl, :] + rbuf_l[s][slot]

            @pl.when(h < N_HOPS - 2)
            def _():
                pl.semaphore_signal(
                    cred_r[s], inc=1,
                    device_id=(left,), device_id_type=pl.DeviceIdType.MESH,
                )
                pl.semaphore_signal(
                    cred_l[s], inc=1,
                    device_id=(right,), device_id_type=pl.DeviceIdType.MESH,
                )

        def wait_credits(h, s):
            @pl.when(h >= 2)
            def _():
                pl.semaphore_wait(cred_r[s], 1)
                pl.semaphore_wait(cred_l[s], 1)

        lax.fori_loop(0, 2, gemm_block, 0)
        for s in range(N_SUB):
            start_hop(0, s, False)
        lax.fori_loop(2, 2 * N_DEV, gemm_block, 0)
        for s in range(N_SUB - 1):
            finish_hop(0, s, False)

        def loop_body(h, _, is_ag, prev_ag):
            wait_credits(h, 0)
            start_hop(h, 0, is_ag)
            finish_hop(h - 1, N_SUB - 1, prev_ag)
            for s in range(1, N_SUB):
                wait_credits(h, s)
                start_hop(h, s, is_ag)
                finish_hop(h, s - 1, is_ag)
            return 0

        ag = N_DEV - 1
        lax.fori_loop(1, ag, lambda h, c: loop_body(h, c, False, False), 0)
        loop_body(ag, 0, True, False)
        lax.fori_loop(ag + 1, N_HOPS, lambda h, c: loop_body(h, c, True, True), 0)

        finish_hop(N_HOPS - 1, N_SUB - 1, True)

    dma_sems = pltpu.SemaphoreType.DMA((2,))
    return pl.pallas_call(
        body,
        out_shape=jax.ShapeDtypeStruct((M, N), jnp.float32),
        in_specs=[
            pl.BlockSpec(memory_space=pltpu.VMEM),
            pl.BlockSpec(memory_space=pltpu.VMEM),
            pl.BlockSpec(memory_space=pltpu.SMEM),
            pl.BlockSpec(memory_space=pltpu.SMEM),
        ],
        out_specs=pl.BlockSpec(memory_space=pltpu.VMEM),
        scratch_shapes=(
            [pltpu.VMEM((2, M_SUB, N), jnp.float32)] * (2 * N_SUB)
            + [dma_sems] * (4 * N_SUB)
            + [pltpu.SemaphoreType.REGULAR] * (2 * N_SUB)
        ),
        compiler_params=pltpu.CompilerParams(
            collective_id=0,
            vmem_limit_bytes=64 * 1024 * 1024,
        ),
    )(x, w_mat, scale_x, scale_w)


# device time: 315667 ns/iter; 1.9750x vs baseline; 1.0037x over previous
import jax
import jax.numpy as jnp
from jax import lax
from jax.experimental import pallas as pl
from jax.experimental.pallas import tpu as pltpu

N_DEV = 4
M = 4096
N = 2048
M_HALF = M // 2
M_CHUNK = M_HALF // N_DEV
N_SUB = 4
M_SUB = M_CHUNK // N_SUB
N_HOPS = 2 * (N_DEV - 1)


def kernel(x, w_mat, scale_x, scale_w):
    k_per = x.shape[1]
    assert x.shape == (M, k_per)
    assert w_mat.shape == (k_per, N)

    def body(x_ref, w_ref, sx_ref, sw_ref, out_ref, *scratch):
        rbuf_r = scratch[:N_SUB]
        rbuf_l = scratch[N_SUB:2 * N_SUB]
        dma = scratch[2 * N_SUB:2 * N_SUB + 4 * N_SUB]
        send_r = dma[0::4]
        recv_r = dma[1::4]
        send_l = dma[2::4]
        recv_l = dma[3::4]
        creds = scratch[2 * N_SUB + 4 * N_SUB:2 * N_SUB + 6 * N_SUB]
        cred_r = creds[0::2]
        cred_l = creds[1::2]
        x_vmem, w_vmem, cp_sems = scratch[2 * N_SUB + 6 * N_SUB:]

        cp_x = pltpu.make_async_copy(x_ref, x_vmem, cp_sems.at[0])
        cp_w = pltpu.make_async_copy(w_ref, w_vmem, cp_sems.at[1])
        cp_x.start()
        cp_w.start()

        my = lax.axis_index("i")
        left = (my + N_DEV - 1) % N_DEV
        right = (my + 1) % N_DEV

        barrier_sem = pltpu.get_barrier_semaphore()
        for nbr in (left, right):
            pl.semaphore_signal(
                barrier_sem, inc=1,
                device_id=(nbr,), device_id_type=pl.DeviceIdType.MESH,
            )
        pl.semaphore_wait(barrier_sem, 2)

        scale = sx_ref[0] * sw_ref[0]

        def gemm_block(i, _):
            blk = (i % 2) * N_DEV + (my + i // 2) % N_DEV
            rows = pl.ds(blk * M_CHUNK, M_CHUNK)
            acc = jnp.dot(
                x_vmem[rows, :],
                w_vmem[:, :],
                preferred_element_type=jnp.int32,
            )
            out_ref[rows, :] = acc.astype(jnp.float32) * scale
            return 0

        def rows_r(chunk, s):
            return pl.ds(chunk * M_CHUNK + s * M_SUB, M_SUB)

        def rows_l(chunk, s):
            return pl.ds(M_HALF + chunk * M_CHUNK + s * M_SUB, M_SUB)

        def make_hop(h, s, is_ag):
            slot = h % 2
            rr = rows_r((my + 2 * N_HOPS - h) % N_DEV, s)
            rl = rows_l((my + h) % N_DEV, s)
            rdma_r = pltpu.make_async_remote_copy(
                src_ref=out_ref.at[rr, :],
                dst_ref=out_ref.at[rr, :] if is_ag else rbuf_r[s].at[slot],
                send_sem=send_r[s].at[slot],
                recv_sem=recv_r[s].at[slot],
                device_id=(right,),
                device_id_type=pl.DeviceIdType.MESH,
            )
            rdma_l = pltpu.make_async_remote_copy(
                src_ref=out_ref.at[rl, :],
                dst_ref=out_ref.at[rl, :] if is_ag else rbuf_l[s].at[slot],
                send_sem=send_l[s].at[slot],
                recv_sem=recv_l[s].at[slot],
                device_id=(left,),
                device_id_type=pl.DeviceIdType.MESH,
            )
            return rdma_r, rdma_l

        def start_hop(h, s, is_ag):
            rdma_r, rdma_l = make_hop(h, s, is_ag)
            rdma_r.start()
            rdma_l.start()

        def finish_hop(h, s, is_ag):
            rdma_r, rdma_l = make_hop(h, s, is_ag)
            rdma_r.wait()
            rdma_l.wait()
            slot = h % 2

            @pl.when(h < N_DEV - 1)
            def _():
                rr = rows_r((my + 2 * N_HOPS - h - 1) % N_DEV, s)
                rl = rows_l((my + h + 1) % N_DEV, s)
                out_ref[rr, :] = out_ref[rr, :] + rbuf_r[s][slot]
                out_ref[rl, :] = out_ref[rl, :] + rbuf_l[s][slot]

            @pl.when(h < N_HOPS - 2)
            def _():
                pl.semaphore_signal(
                    cred_r[s], inc=1,
                    device_id=(left,), device_id_type=pl.DeviceIdType.MESH,
                )
                pl.semaphore_signal(
                    cred_l[s], inc=1,
                    device_id=(right,), device_id_type=pl.DeviceIdType.MESH,
                )

        def wait_credits(h, s):
            @pl.when(h >= 2)
            def _():
                pl.semaphore_wait(cred_r[s], 1)
                pl.semaphore_wait(cred_l[s], 1)

        cp_x.wait()
        cp_w.wait()
        lax.fori_loop(0, 2, gemm_block, 0)
        for s in range(N_SUB):
            start_hop(0, s, False)
        lax.fori_loop(2, 2 * N_DEV, gemm_block, 0)
        for s in range(N_SUB - 1):
            finish_hop(0, s, False)

        def loop_body(h, _, is_ag, prev_ag):
            wait_credits(h, 0)
            start_hop(h, 0, is_ag)
            finish_hop(h - 1, N_SUB - 1, prev_ag)
            for s in range(1, N_SUB):
                wait_credits(h, s)
                start_hop(h, s, is_ag)
                finish_hop(h, s - 1, is_ag)
            return 0

        ag = N_DEV - 1
        lax.fori_loop(1, ag, lambda h, c: loop_body(h, c, False, False), 0)
        loop_body(ag, 0, True, False)
        lax.fori_loop(ag + 1, N_HOPS, lambda h, c: loop_body(h, c, True, True), 0)

        finish_hop(N_HOPS - 1, N_SUB - 1, True)

    dma_sems = pltpu.SemaphoreType.DMA((2,))
    return pl.pallas_call(
        body,
        out_shape=jax.ShapeDtypeStruct((M, N), jnp.float32),
        in_specs=[
            pl.BlockSpec(memory_space=pl.ANY),
            pl.BlockSpec(memory_space=pl.ANY),
            pl.BlockSpec(memory_space=pltpu.SMEM),
            pl.BlockSpec(memory_space=pltpu.SMEM),
        ],
        out_specs=pl.BlockSpec(memory_space=pltpu.VMEM),
        scratch_shapes=(
            [pltpu.VMEM((2, M_SUB, N), jnp.float32)] * (2 * N_SUB)
            + [dma_sems] * (4 * N_SUB)
            + [pltpu.SemaphoreType.REGULAR] * (2 * N_SUB)
            + [
                pltpu.VMEM((M, 1024), jnp.int8),
                pltpu.VMEM((1024, N), jnp.int8),
                pltpu.SemaphoreType.DMA((2,)),
            ]
        ),
        compiler_params=pltpu.CompilerParams(
            collective_id=0,
            vmem_limit_bytes=64 * 1024 * 1024,
        ),
    )(x, w_mat, scale_x, scale_w)
